# baseline (speedup 1.0000x reference)
import functools

import jax
import jax.numpy as jnp
from jax import lax
from jax.experimental import pallas as pl
from jax.experimental.pallas import tpu as pltpu
from jax.experimental.pallas import tpu_sc as plsc

B = 16384
D = 128
V_SMALL = 1000
W_OFF = 4 * V_SMALL

NC = 2
NS = 16
L = 16
NW = NC * NS
BPW = B // NW
NG = BPW // 128
GPC = 128 // L


def _aux_body(t0_ref, t1_ref, t2_ref, t3_ref, w_ref, b_ref, out_ref):
    w = w_ref[...]
    out_ref[pl.ds(0, V_SMALL), :] = jnp.dot(
        t0_ref[...], w, preferred_element_type=jnp.float32) + b_ref[...]
    out_ref[pl.ds(V_SMALL, V_SMALL), :] = jnp.dot(
        t1_ref[...], w, preferred_element_type=jnp.float32)
    out_ref[pl.ds(2 * V_SMALL, V_SMALL), :] = jnp.dot(
        t2_ref[...], w, preferred_element_type=jnp.float32)
    out_ref[pl.ds(3 * V_SMALL, V_SMALL), :] = jnp.dot(
        t3_ref[...], w, preferred_element_type=jnp.float32)
    out_ref[pl.ds(W_OFF, D), :] = w


def _sc_body(didx_hbm, li_hbm, ri_hbm, ai_hbm, ui_hbm,
             desc_hbm, aux_hbm, out_hbm,
             idxd, rows, sidx, aux, ov, sem_i, sem_r, sem_a):
    wid = lax.axis_index("s") * NC + lax.axis_index("c")
    base = wid * BPW
    riota = lax.iota(jnp.int32, L)

    cp_idx = pltpu.async_copy(didx_hbm.at[pl.ds(wid * NG, NG)], idxd, sem_i)
    cps = [pltpu.async_copy(h.at[pl.ds(base, BPW)], sidx.at[q], sem_a)
           for q, h in enumerate((li_hbm, ri_hbm, ai_hbm, ui_hbm))]
    cp_aux = pltpu.async_copy(aux_hbm, aux, sem_a)

    cp_idx.wait()
    gathers = [
        pltpu.async_copy(desc_hbm.at[idxd.at[j]],
                         rows.at[pl.ds(j * 128, 128)], sem_r)
        for j in range(NG)
    ]

    for cp in cps:
        cp.wait()
    cp_aux.wait()

    for g in range(BPW // L):
        s = pl.ds(g * L, L)
        acc = plsc.load_gather(aux, [sidx[0, s]])
        acc = acc + plsc.load_gather(aux, [sidx[1, s] + V_SMALL])
        acc = acc + plsc.load_gather(aux, [sidx[2, s] + 2 * V_SMALL])
        acc = acc + plsc.load_gather(aux, [sidx[3, s] + 3 * V_SMALL])
        ov[s] = acc

    zero = jnp.zeros((L,), jnp.float32)
    for j in range(NG):
        gathers[j].wait()
        rvecs = [riota + (j * 128 + g * L) for g in range(GPC)]

        def dot_body(d, carry, rvecs=rvecs):
            colv = carry[0]
            wdiag = plsc.load_gather(aux, [colv + W_OFF])
            accs = [acc + plsc.load_gather(rows, [rvecs[g], colv]) * wdiag
                    for g, acc in enumerate(carry[1])]
            return ((colv + 1) & (D - 1), accs)

        _, accs = lax.fori_loop(0, D, dot_body, (riota, [zero] * GPC))
        for g in range(GPC):
            s = pl.ds(j * 128 + g * L, L)
            ov[s] = ov[s] + accs[g]

    pltpu.sync_copy(ov, out_hbm.at[pl.ds(base, BPW)])


def kernel(desc_idx, lang_idx, rel_idx, avg_idx, run_idx,
           desc_table, lang_table, rel_table, avg_table, run_table,
           W_out, b_out):
    w128 = jnp.concatenate([W_out, W_out], axis=0)

    aux = pl.pallas_call(
        _aux_body,
        out_shape=jax.ShapeDtypeStruct((W_OFF + D, 1), jnp.float32),
    )(lang_table, rel_table, avg_table, run_table, w128, b_out.reshape(1, 1))

    sc = pl.kernel(
        _sc_body,
        out_type=jax.ShapeDtypeStruct((B,), jnp.float32),
        mesh=plsc.VectorSubcoreMesh(core_axis_name="c", subcore_axis_name="s"),
        compiler_params=pltpu.CompilerParams(needs_layout_passes=False),
        scratch_types=[
            pltpu.VMEM((NG, 128), jnp.int32),
            pltpu.VMEM((BPW, D), jnp.float32),
            pltpu.VMEM((4, BPW), jnp.int32),
            pltpu.VMEM((W_OFF + D,), jnp.float32),
            pltpu.VMEM((BPW,), jnp.float32),
            pltpu.SemaphoreType.DMA,
            pltpu.SemaphoreType.DMA,
            pltpu.SemaphoreType.DMA,
        ],
    )

    out = sc(desc_idx.reshape(B // 128, 128), lang_idx, rel_idx, avg_idx,
             run_idx, desc_table, aux.reshape(W_OFF + D))
    return out.reshape(B, 1)
